# Initial kernel scaffold; baseline (speedup 1.0000x reference)
import jax
import jax.numpy as jnp
from jax.experimental import pallas as pl


def kernel(x, Wg, We, be):
    raise NotImplementedError("write your pallas kernel here")


# speedup vs baseline: 3.1314x; 3.1314x over previous
import functools

import jax
import jax.numpy as jnp
from jax.experimental import pallas as pl
from jax.experimental.pallas import tpu as pltpu

NUM_EXPERTS = 8
TOP_K = 2
D_MODEL = 768
SEQ = 2048

TOK_TILE = 256


def _moe_dense_body(x_ref, wg_ref, we_ref, be_ref, out_ref):
    xt = x_ref[...]
    T = xt.shape[0]

    logits = jax.lax.dot_general(
        xt, wg_ref[...], (((1,), (1,)), ((), ())),
        preferred_element_type=jnp.float32)

    iota = jax.lax.broadcasted_iota(jnp.int32, (T, NUM_EXPERTS), 1)
    m1 = jnp.max(logits, axis=1, keepdims=True)
    a1 = jnp.min(jnp.where(logits >= m1, iota, NUM_EXPERTS), axis=1,
                 keepdims=True)
    masked = jnp.where(iota == a1, -jnp.inf, logits)
    m2 = jnp.max(masked, axis=1, keepdims=True)
    a2 = jnp.min(jnp.where(masked >= m2, iota, NUM_EXPERTS), axis=1,
                 keepdims=True)

    w1 = 1.0 / (1.0 + jnp.exp(m2 - m1))
    w2 = 1.0 - w1
    cw = jnp.where(iota == a1, w1, 0.0) + jnp.where(iota == a2, w2, 0.0)

    acc = jax.lax.dot_general(
        cw, be_ref[...], (((1,), (0,)), ((), ())),
        preferred_element_type=jnp.float32)
    for e in range(NUM_EXPERTS):
        ye = jax.lax.dot_general(
            xt, we_ref[e], (((1,), (1,)), ((), ())),
            preferred_element_type=jnp.float32)
        acc = acc + cw[:, e:e + 1] * ye
    out_ref[...] = acc


@jax.jit
def _moe_dense(x2d, Wg, We, be):
    n_tiles = SEQ // TOK_TILE
    return pl.pallas_call(
        _moe_dense_body,
        grid=(n_tiles,),
        in_specs=[
            pl.BlockSpec((TOK_TILE, D_MODEL), lambda i: (i, 0)),
            pl.BlockSpec((NUM_EXPERTS, D_MODEL), lambda i: (0, 0)),
            pl.BlockSpec((NUM_EXPERTS, D_MODEL, D_MODEL), lambda i: (0, 0, 0)),
            pl.BlockSpec((NUM_EXPERTS, D_MODEL), lambda i: (0, 0)),
        ],
        out_specs=pl.BlockSpec((TOK_TILE, D_MODEL), lambda i: (i, 0)),
        out_shape=jax.ShapeDtypeStruct((SEQ, D_MODEL), jnp.float32),
        compiler_params=pltpu.CompilerParams(
            dimension_semantics=("arbitrary",)),
    )(x2d, Wg, We, be)


def kernel(x, Wg, We, be):
    B, S, D = x.shape
    out = _moe_dense(x.reshape(S, D), Wg, We, be)
    return out.reshape(B, S, D)
